# Initial kernel scaffold; baseline (speedup 1.0000x reference)
import jax
import jax.numpy as jnp
from jax.experimental import pallas as pl


def kernel(pred, target):
    raise NotImplementedError("write your pallas kernel here")


# speedup vs baseline: 2.0223x; 2.0223x over previous
import math

import jax
import jax.numpy as jnp
from jax.experimental import pallas as pl
from jax.experimental.pallas import tpu as pltpu

SMOOTHING = 0.1
CONF = 1.0 - SMOOTHING


def _loss_kernel(x_ref, tgt_ref, out_ref):
    x = x_ref[...]
    r, c = x.shape

    m = jnp.max(x, axis=1, keepdims=True)
    s = jnp.sum(jnp.exp(x - m), axis=1, keepdims=True)
    lse = m + jnp.log(s)
    rs = jnp.sum(x, axis=1, keepdims=True)

    cols = jax.lax.broadcasted_iota(jnp.int32, (r, c), 1)
    g = jnp.sum(jnp.where(cols == tgt_ref[...], x, 0.0),
                axis=1, keepdims=True)

    eps = SMOOTHING / (c - 1)
    kl_coef = eps * c + CONF - eps
    term = eps * rs - kl_coef * lse + (CONF - eps) * g
    out_ref[...] = jnp.sum(term).reshape(1, 1, 1)


def kernel(pred, target):
    b, c = pred.shape
    r = 32
    nb = b // r

    tgt = target.astype(jnp.int32).reshape(b, 1)

    partials = pl.pallas_call(
        _loss_kernel,
        grid=(nb,),
        in_specs=[
            pl.BlockSpec((r, c), lambda i: (i, 0)),
            pl.BlockSpec((r, 1), lambda i: (i, 0)),
        ],
        out_specs=pl.BlockSpec((1, 1, 1), lambda i: (i, 0, 0)),
        out_shape=jax.ShapeDtypeStruct((nb, 1, 1), jnp.float32),
        compiler_params=pltpu.CompilerParams(
            dimension_semantics=("parallel",),
        ),
    )(pred, tgt)

    eps = SMOOTHING / (c - 1)
    const = (c - 1) * eps * math.log(eps) + CONF * math.log(CONF)
    return (const - jnp.sum(partials) / b).astype(jnp.float32)
